# baseline (speedup 1.0000x reference)
import functools

import jax
import jax.numpy as jnp
from jax.experimental import pallas as pl
from jax.experimental.pallas import tpu as pltpu


def _knn_body(q_ref, k_ref, q2_ref, k2_ref,
              minval_ref, minidx_ref, sstar_ref, sidx_ref,
              qm2_s, ridx_s, run_min, run_idx, *, kb_size, nk):
    kb = pl.program_id(0)

    @pl.when(kb == 0)
    def _():
        qm2_s[...] = q_ref[...] * jnp.float32(-2.0)
        ridx_s[...] = jax.lax.broadcasted_iota(
            jnp.int32, (kb_size, 1), 0).astype(jnp.float32)

    k = k_ref[...]
    qk = jax.lax.dot_general(
        k, qm2_s[...], dimension_numbers=(((1,), (1,)), ((), ())),
        preferred_element_type=jnp.float32)
    d2 = (q2_ref[...] + k2_ref[...]) + qk

    m2 = jnp.min(d2, axis=0, keepdims=True)
    cm = jnp.maximum(m2, 1e-12)
    s = jnp.sqrt(cm)
    yi = jax.lax.bitcast_convert_type(s * s, jnp.int32)
    h = cm
    for step in range(6):
        yk = jax.lax.bitcast_convert_type(yi + step, jnp.float32)
        h = jnp.maximum(h, jnp.where(jnp.sqrt(yk) == s, yk, cm))

    big = jnp.float32(kb_size)
    li = (jnp.min(jnp.where(d2 <= h, ridx_s[...], big), axis=0, keepdims=True)
          + (kb * kb_size).astype(jnp.float32))

    @pl.when(kb == 0)
    def _():
        run_min[...] = s
        run_idx[...] = li

    @pl.when(kb > 0)
    def _():
        better = s < run_min[...]
        run_min[...] = jnp.where(better, s, run_min[...])
        run_idx[...] = jnp.where(better, li, run_idx[...])

    @pl.when(kb == nk - 1)
    def _():
        mv = run_min[...]
        q_n = mv.shape[1]
        minval_ref[...] = mv.reshape(1, 1, q_n)
        minidx_ref[...] = run_idx[...].astype(jnp.int32).reshape(q_n)
        st = jnp.max(mv, axis=1, keepdims=True)
        sstar_ref[...] = st
        qiota = jax.lax.broadcasted_iota(
            jnp.int32, mv.shape, 1).astype(jnp.float32)
        sidx_ref[...] = jnp.min(
            jnp.where(mv == st, qiota, jnp.float32(q_n)),
            axis=1, keepdims=True).astype(jnp.int32)


def kernel(patch, patch_lib):
    q_n, d = patch.shape
    k_n, _ = patch_lib.shape
    kb_size = 4096
    nk = k_n // kb_size

    q2 = jnp.sum(patch * patch, axis=1)[None, :]
    k2 = jnp.sum(patch_lib * patch_lib, axis=1)[:, None]

    body = functools.partial(_knn_body, kb_size=kb_size, nk=nk)
    s_map, min_idx, sstar, sidx = pl.pallas_call(
        body,
        grid=(nk,),
        in_specs=[
            pl.BlockSpec((q_n, d), lambda i: (0, 0)),
            pl.BlockSpec((kb_size, d), lambda i: (i, 0)),
            pl.BlockSpec((1, q_n), lambda i: (0, 0)),
            pl.BlockSpec((kb_size, 1), lambda i: (i, 0)),
        ],
        out_specs=[
            pl.BlockSpec((1, 1, q_n), lambda i: (0, 0, 0)),
            pl.BlockSpec((q_n,), lambda i: (0,)),
            pl.BlockSpec((1, 1), lambda i: (0, 0)),
            pl.BlockSpec((1, 1), lambda i: (0, 0)),
        ],
        out_shape=[
            jax.ShapeDtypeStruct((1, 1, q_n), jnp.float32),
            jax.ShapeDtypeStruct((q_n,), jnp.int32),
            jax.ShapeDtypeStruct((1, 1), jnp.float32),
            jax.ShapeDtypeStruct((1, 1), jnp.int32),
        ],
        scratch_shapes=[
            pltpu.VMEM((q_n, d), jnp.float32),
            pltpu.VMEM((kb_size, 1), jnp.float32),
            pltpu.VMEM((1, q_n), jnp.float32),
            pltpu.VMEM((1, q_n), jnp.float32),
        ],
    )(patch, patch_lib, q2, k2)

    s_star = sstar.reshape(())
    s_idx = sidx.reshape(())
    return (s_map, s_star, s_idx, min_idx)


# speedup vs baseline: 1.0024x; 1.0024x over previous
import functools

import jax
import jax.numpy as jnp
from jax.experimental import pallas as pl
from jax.experimental.pallas import tpu as pltpu


def _knn_body(q_ref, k_ref, q2_ref, k2_ref,
              minval_ref, minidx_ref, sstar_ref, sidx_ref,
              qm2_s, ridx_s, run_min, run_idx, *, kb_size, nk):
    kb = pl.program_id(0)

    @pl.when(kb == 0)
    def _():
        qm2_s[...] = q_ref[...] * jnp.float32(-2.0)
        ridx_s[...] = jax.lax.broadcasted_iota(
            jnp.int32, (kb_size, 1), 0).astype(jnp.float32)

    k = k_ref[...]
    qk = jax.lax.dot_general(
        k, qm2_s[...], dimension_numbers=(((1,), (1,)), ((), ())),
        preferred_element_type=jnp.float32)
    d2 = (q2_ref[...] + k2_ref[...]) + qk

    m2 = jnp.min(d2, axis=0, keepdims=True)
    cm = jnp.maximum(m2, 1e-12)
    s = jnp.sqrt(cm)
    yi = jax.lax.bitcast_convert_type(s * s, jnp.int32)
    h = cm
    for step in range(6):
        yk = jax.lax.bitcast_convert_type(yi + step, jnp.float32)
        h = jnp.maximum(h, jnp.where(jnp.sqrt(yk) == s, yk, cm))

    big = jnp.float32(kb_size)
    li = (jnp.min(jnp.where(d2 <= h, ridx_s[...], big), axis=0, keepdims=True)
          + (kb * kb_size).astype(jnp.float32))

    @pl.when(kb == 0)
    def _():
        run_min[...] = s
        run_idx[...] = li

    @pl.when(kb > 0)
    def _():
        better = s < run_min[...]
        run_min[...] = jnp.where(better, s, run_min[...])
        run_idx[...] = jnp.where(better, li, run_idx[...])

    @pl.when(kb == nk - 1)
    def _():
        mv = run_min[...]
        q_n = mv.shape[1]
        minval_ref[...] = mv.reshape(1, 1, q_n)
        minidx_ref[...] = run_idx[...].astype(jnp.int32).reshape(q_n)
        st = jnp.max(mv, axis=1, keepdims=True)
        sstar_ref[...] = st
        qiota = jax.lax.broadcasted_iota(
            jnp.int32, mv.shape, 1).astype(jnp.float32)
        sidx_ref[...] = jnp.min(
            jnp.where(mv == st, qiota, jnp.float32(q_n)),
            axis=1, keepdims=True).astype(jnp.int32)


def kernel(patch, patch_lib):
    q_n, d = patch.shape
    k_n, _ = patch_lib.shape
    kb_size = 1024
    nk = k_n // kb_size

    q2 = jnp.sum(patch * patch, axis=1)[None, :]
    k2 = jnp.sum(patch_lib * patch_lib, axis=1)[:, None]

    body = functools.partial(_knn_body, kb_size=kb_size, nk=nk)
    s_map, min_idx, sstar, sidx = pl.pallas_call(
        body,
        grid=(nk,),
        in_specs=[
            pl.BlockSpec((q_n, d), lambda i: (0, 0)),
            pl.BlockSpec((kb_size, d), lambda i: (i, 0)),
            pl.BlockSpec((1, q_n), lambda i: (0, 0)),
            pl.BlockSpec((kb_size, 1), lambda i: (i, 0)),
        ],
        out_specs=[
            pl.BlockSpec((1, 1, q_n), lambda i: (0, 0, 0)),
            pl.BlockSpec((q_n,), lambda i: (0,)),
            pl.BlockSpec((1, 1), lambda i: (0, 0)),
            pl.BlockSpec((1, 1), lambda i: (0, 0)),
        ],
        out_shape=[
            jax.ShapeDtypeStruct((1, 1, q_n), jnp.float32),
            jax.ShapeDtypeStruct((q_n,), jnp.int32),
            jax.ShapeDtypeStruct((1, 1), jnp.float32),
            jax.ShapeDtypeStruct((1, 1), jnp.int32),
        ],
        scratch_shapes=[
            pltpu.VMEM((q_n, d), jnp.float32),
            pltpu.VMEM((kb_size, 1), jnp.float32),
            pltpu.VMEM((1, q_n), jnp.float32),
            pltpu.VMEM((1, q_n), jnp.float32),
        ],
    )(patch, patch_lib, q2, k2)

    s_star = sstar.reshape(())
    s_idx = sidx.reshape(())
    return (s_map, s_star, s_idx, min_idx)


# speedup vs baseline: 1.0318x; 1.0294x over previous
import functools

import jax
import jax.numpy as jnp
from jax.experimental import pallas as pl
from jax.experimental.pallas import tpu as pltpu


def _knn_body(q_ref, k_ref, q2_ref, k2_ref,
              minval_ref, minidx_ref, sstar_ref, sidx_ref,
              qm2_s, ridx_s, run_min, run_idx, *, kb_size, nk):
    kb = pl.program_id(0)

    @pl.when(kb == 0)
    def _():
        qm2_s[...] = q_ref[...] * jnp.float32(-2.0)
        ridx_s[...] = jax.lax.broadcasted_iota(
            jnp.int32, (kb_size, 1), 0).astype(jnp.float32)

    k = k_ref[...]
    qk = jax.lax.dot_general(
        k, qm2_s[...], dimension_numbers=(((1,), (1,)), ((), ())),
        preferred_element_type=jnp.float32)
    d2 = (q2_ref[...] + k2_ref[...]) + qk

    m2 = jnp.min(d2, axis=0, keepdims=True)
    cm = jnp.maximum(m2, 1e-12)
    s = jnp.sqrt(cm)
    yi = jax.lax.bitcast_convert_type(s * s, jnp.int32)
    h = cm
    for step in range(6):
        yk = jax.lax.bitcast_convert_type(yi + step, jnp.float32)
        h = jnp.maximum(h, jnp.where(jnp.sqrt(yk) == s, yk, cm))

    big = jnp.float32(kb_size)
    li = (jnp.min(jnp.where(d2 <= h, ridx_s[...], big), axis=0, keepdims=True)
          + (kb * kb_size).astype(jnp.float32))

    @pl.when(kb == 0)
    def _():
        run_min[...] = s
        run_idx[...] = li

    @pl.when(kb > 0)
    def _():
        better = s < run_min[...]
        run_min[...] = jnp.where(better, s, run_min[...])
        run_idx[...] = jnp.where(better, li, run_idx[...])

    @pl.when(kb == nk - 1)
    def _():
        mv = run_min[...]
        q_n = mv.shape[1]
        minval_ref[...] = mv.reshape(1, 1, q_n)
        minidx_ref[...] = run_idx[...].astype(jnp.int32).reshape(q_n)
        st = jnp.max(mv, axis=1, keepdims=True)
        sstar_ref[...] = st
        qiota = jax.lax.broadcasted_iota(
            jnp.int32, mv.shape, 1).astype(jnp.float32)
        sidx_ref[...] = jnp.min(
            jnp.where(mv == st, qiota, jnp.float32(q_n)),
            axis=1, keepdims=True).astype(jnp.int32)


def kernel(patch, patch_lib):
    q_n, d = patch.shape
    k_n, _ = patch_lib.shape
    kb_size = 2048
    nk = k_n // kb_size

    q2 = jnp.sum(patch * patch, axis=1)[None, :]
    k2 = jnp.sum(patch_lib * patch_lib, axis=1)[:, None]

    body = functools.partial(_knn_body, kb_size=kb_size, nk=nk)
    s_map, min_idx, sstar, sidx = pl.pallas_call(
        body,
        grid=(nk,),
        in_specs=[
            pl.BlockSpec((q_n, d), lambda i: (0, 0)),
            pl.BlockSpec((kb_size, d), lambda i: (i, 0)),
            pl.BlockSpec((1, q_n), lambda i: (0, 0)),
            pl.BlockSpec((kb_size, 1), lambda i: (i, 0)),
        ],
        out_specs=[
            pl.BlockSpec((1, 1, q_n), lambda i: (0, 0, 0)),
            pl.BlockSpec((q_n,), lambda i: (0,)),
            pl.BlockSpec((1, 1), lambda i: (0, 0)),
            pl.BlockSpec((1, 1), lambda i: (0, 0)),
        ],
        out_shape=[
            jax.ShapeDtypeStruct((1, 1, q_n), jnp.float32),
            jax.ShapeDtypeStruct((q_n,), jnp.int32),
            jax.ShapeDtypeStruct((1, 1), jnp.float32),
            jax.ShapeDtypeStruct((1, 1), jnp.int32),
        ],
        scratch_shapes=[
            pltpu.VMEM((q_n, d), jnp.float32),
            pltpu.VMEM((kb_size, 1), jnp.float32),
            pltpu.VMEM((1, q_n), jnp.float32),
            pltpu.VMEM((1, q_n), jnp.float32),
        ],
    )(patch, patch_lib, q2, k2)

    s_star = sstar.reshape(())
    s_idx = sidx.reshape(())
    return (s_map, s_star, s_idx, min_idx)
